# baseline (speedup 1.0000x reference)
import functools

import jax
import jax.numpy as jnp
from jax import lax
from jax.experimental import pallas as pl
from jax.experimental.pallas import tpu as pltpu
from jax.experimental.pallas import tpu_sc as plsc

NUM_USERS = 1000000
NUM_EVENTS = 1000000
EMBED_DIM = 64
BATCH = 16384

L = 16


def _make_sc_kernel():
    info = plsc.get_sparse_core_info()
    nc, ns = info.num_cores, info.num_subcores
    nw = nc * ns
    bpw = BATCH // nw
    nblk = bpw // L

    mesh = plsc.VectorSubcoreMesh(core_axis_name="c", subcore_axis_name="s")

    @functools.partial(
        pl.kernel,
        mesh=mesh,
        out_type=jax.ShapeDtypeStruct((BATCH,), jnp.float32),
        scratch_types=[
            pltpu.VMEM((bpw,), jnp.int32),
            pltpu.VMEM((bpw,), jnp.int32),
            pltpu.VMEM((bpw, EMBED_DIM), jnp.float32),
            pltpu.VMEM((bpw, EMBED_DIM), jnp.float32),
            pltpu.VMEM((bpw,), jnp.float32),
            pltpu.VMEM((bpw,), jnp.float32),
            pltpu.VMEM((L,), jnp.float32),
            pltpu.VMEM((bpw,), jnp.float32),
            pltpu.SemaphoreType.DMA,
            pltpu.SemaphoreType.DMA,
            pltpu.SemaphoreType.DMA,
            pltpu.SemaphoreType.DMA,
        ],
        compiler_params=pltpu.CompilerParams(
            needs_layout_passes=False, use_tc_tiling_on_sc=False),
    )
    def sc_kernel(uid_hbm, eid_hbm, ut_hbm, et_hbm, ub_hbm, eb_hbm, gb_hbm,
                  out_hbm, uid_v, eid_v, u_rows, e_rows, ub_v, eb_v, gb_v,
                  scores_v, sem0, sem1, sem2, sem3):
        wid = lax.axis_index("s") * nc + lax.axis_index("c")
        base = wid * bpw

        pltpu.sync_copy(uid_hbm.at[pl.ds(base, bpw)], uid_v)
        pltpu.sync_copy(eid_hbm.at[pl.ds(base, bpw)], eid_v)
        cu = pltpu.async_copy(ut_hbm.at[uid_v], u_rows, sem0)
        ce = pltpu.async_copy(et_hbm.at[eid_v], e_rows, sem1)
        cub = pltpu.async_copy(ub_hbm.at[uid_v], ub_v, sem2)
        ceb = pltpu.async_copy(eb_hbm.at[eid_v], eb_v, sem3)
        pltpu.sync_copy(gb_hbm.at[pl.ds(0, 1)], gb_v.at[pl.ds(0, 1)])
        cu.wait()
        ce.wait()
        cub.wait()
        ceb.wait()

        gb = gb_v[pl.ds(0, L)][0]
        lane = lax.iota(jnp.int32, L)

        def block(k, _):
            row = jnp.full((L,), k * L, jnp.int32) + lane
            acc = jnp.full((L,), gb, jnp.float32)
            for d in range(EMBED_DIM):
                col = jnp.full((L,), d, jnp.int32)
                gu = plsc.load_gather(u_rows, [row, col])
                ge = plsc.load_gather(e_rows, [row, col])
                acc = acc + gu * ge
            acc = acc + ub_v[pl.ds(k * L, L)]
            acc = acc + eb_v[pl.ds(k * L, L)]
            scores_v[pl.ds(k * L, L)] = acc
            return _

        lax.fori_loop(0, nblk, block, None)
        pltpu.sync_copy(scores_v, out_hbm.at[pl.ds(base, bpw)])

    return sc_kernel


_sc_kernel = _make_sc_kernel()


def kernel(user_ids, event_ids, user_table, event_table, user_bias,
           event_bias, global_bias):
    uid = user_ids.astype(jnp.int32)
    eid = event_ids.astype(jnp.int32)
    return _sc_kernel(uid, eid, user_table, event_table,
                      user_bias.reshape(-1), event_bias.reshape(-1),
                      global_bias)
